# baseline (speedup 1.0000x reference)
import jax
import jax.numpy as jnp
from jax.experimental import pallas as pl

DIM = 128
B = 1024
L = 20
S = 12
T = 1.0
LRELU = 0.2


def _leaky(x, slope):
    return jnp.where(x >= 0, x, slope * x)


def kernel(inputs, adj, mask_item, item, embedding, a0, a1, a2, a3, g_w1, g_w2, g_w3, num_w, adj_all):
    h = jnp.take(embedding, inputs, axis=0)

    a_input = h[:, :, None, :] * h[:, None, :, :]
    e0 = _leaky(jnp.squeeze(a_input @ a0, -1), LRELU)
    e1 = _leaky(jnp.squeeze(a_input @ a1, -1), LRELU)
    e2 = _leaky(jnp.squeeze(a_input @ a2, -1), LRELU)
    e3 = _leaky(jnp.squeeze(a_input @ a3, -1), LRELU)
    neg = jnp.full_like(e0, -9e15)
    att = jnp.where(adj == 1, e0, neg)
    att = jnp.where(adj == 2, e1, att)
    att = jnp.where(adj == 3, e2, att)
    att = jnp.where(adj == 4, e3, att)
    att = jax.nn.softmax(att, axis=-1)
    h_local = att @ h

    flat = inputs.reshape(-1)
    neighbors = jnp.take(adj_all, flat, axis=0).reshape(B, L * S)
    nw = jnp.take(num_w, flat, axis=0).reshape(B, L, S)
    neigh_vec = jnp.take(embedding, neighbors, axis=0).reshape(B, L, S, DIM)

    maskf = mask_item.astype(jnp.float32)
    item_emb = jnp.take(embedding, item, axis=0) * maskf[..., None]
    sum_item_emb = jnp.sum(item_emb, axis=1) / jnp.sum(maskf, axis=-1, keepdims=True)

    ev = sum_item_emb[:, None, None, :]
    feat = jnp.concatenate([ev * neigh_vec, nw[..., None]], axis=-1)
    al = _leaky(feat @ g_w1, 0.2)
    al = jnp.squeeze(al @ g_w2, -1)
    al = jax.nn.softmax(al / T, axis=-1)[..., None]
    agg = jnp.sum(al * neigh_vec, axis=-2)
    out = jnp.concatenate([h, agg], axis=-1) @ g_w3
    s_global = jax.nn.relu(out.reshape(B, L, DIM))

    output = h_local + s_global
    return (output, s_global)


# speedup vs baseline: 1.9324x; 1.9324x over previous
import jax
import jax.numpy as jnp
from jax.experimental import pallas as pl
from jax.experimental.pallas import tpu as pltpu
from jax.experimental.pallas import tpu_sc as plsc

DIM = 128
B = 1024
L = 20
S = 12
LRELU = 0.2
NEG = -9e15

_GW = 128


def _leaky(x, slope=LRELU):
    return jnp.where(x >= 0, x, slope * x)


def _sc_gather(table, idx, width):
    n = idx.shape[1]
    mesh = plsc.VectorSubcoreMesh(core_axis_name="core", subcore_axis_name="subcore")

    @pl.kernel(out_type=jax.ShapeDtypeStruct((n, width), table.dtype), mesh=mesh)
    def gather_kernel(tbl_hbm, i_hbm, o_hbm):
        def body(i_vmem, o_vmem):
            pltpu.sync_copy(tbl_hbm.at[i_vmem.at[0]], o_vmem)

        pltpu.emit_pipeline(
            body,
            grid=(n // _GW,),
            in_specs=[pl.BlockSpec((1, _GW), lambda i: (0, i))],
            out_specs=[pl.BlockSpec((_GW, width), lambda i: (i, 0))],
            core_axis_name=("core", "subcore"),
            dimension_semantics=(pltpu.PARALLEL,),
        )(i_hbm, o_hbm)

    return gather_kernel(table, idx)


def _local_body(h_ref, adj_ref, item_ref, mask_ref, a_ref, hl_ref, ev_ref):
    bb = h_ref.shape[0]
    maskf = mask_ref[...].astype(jnp.float32)
    item_v = item_ref[...]
    s = jnp.sum(item_v * maskf[..., None], axis=1)
    ev_ref[...] = s / jnp.sum(maskf, axis=1, keepdims=True)

    a = a_ref[...]
    for i in range(bb):
        hb = h_ref[i]
        hs = jnp.concatenate(
            [hb * a[0:1], hb * a[1:2], hb * a[2:3], hb * a[3:4]], axis=0
        )
        e = jax.lax.dot_general(
            hs, hb, (((1,), (1,)), ((), ())), preferred_element_type=jnp.float32
        )
        e = _leaky(e)
        e4 = e.reshape(4, L, L)
        adjb = adj_ref[i]
        att = jnp.where(adjb == 1, e4[0], jnp.float32(NEG))
        att = jnp.where(adjb == 2, e4[1], att)
        att = jnp.where(adjb == 3, e4[2], att)
        att = jnp.where(adjb == 4, e4[3], att)
        att = att - jnp.max(att, axis=-1, keepdims=True)
        p = jnp.exp(att)
        att = p / jnp.sum(p, axis=-1, keepdims=True)
        hl_ref[i] = jax.lax.dot_general(
            att, hb, (((1,), (0,)), ((), ())), preferred_element_type=jnp.float32
        )


def _global_body(nv_ref, nw_ref, ev_ref, h_ref, hl_ref,
                 w1a_ref, w1b_ref, w2_ref, w3h_ref, w3a_ref,
                 out_ref, sg_ref):
    bb = nv_ref.shape[0]
    m = bb * L * S
    nv = nv_ref[...]
    ev = ev_ref[...]
    scaled = (nv * ev[:, None, :]).reshape(m, DIM)
    alp = jax.lax.dot_general(
        scaled, w1a_ref[...], (((1,), (0,)), ((), ())),
        preferred_element_type=jnp.float32,
    )
    alp = alp + nw_ref[...] * w1b_ref[0:1, :]
    alp = _leaky(alp)
    al2 = jax.lax.dot_general(
        alp, w2_ref[...], (((1,), (0,)), ((), ())),
        preferred_element_type=jnp.float32,
    )
    al3 = al2.reshape(bb * L, S, DIM)
    mx = jnp.max(al3, axis=1, keepdims=True)
    w = jnp.exp(al3 - mx)
    w = w / jnp.sum(w, axis=1, keepdims=True)
    nv3 = nv.reshape(bb * L, S, DIM)
    agg = jnp.sum(w * nv3, axis=1)
    h2 = h_ref[...].reshape(bb * L, DIM)
    out = jax.lax.dot_general(
        h2, w3h_ref[...], (((1,), (0,)), ((), ())),
        preferred_element_type=jnp.float32,
    ) + jax.lax.dot_general(
        agg, w3a_ref[...], (((1,), (0,)), ((), ())),
        preferred_element_type=jnp.float32,
    )
    sg = jnp.maximum(out, 0.0).reshape(bb, L, DIM)
    sg_ref[...] = sg
    out_ref[...] = hl_ref[...] + sg


def kernel(inputs, adj, mask_item, item, embedding, a0, a1, a2, a3, g_w1, g_w2, g_w3, num_w, adj_all):
    f32 = jnp.float32
    flat = inputs.reshape(1, B * L).astype(jnp.int32)
    item_flat = item.reshape(1, B * L).astype(jnp.int32)

    nw_bits = jax.lax.bitcast_convert_type(num_w, jnp.int32)
    combined = jnp.concatenate(
        [adj_all.astype(jnp.int32), nw_bits,
         jnp.zeros((adj_all.shape[0], 128 - 2 * S), jnp.int32)], axis=1)

    h_rows = _sc_gather(embedding, flat, DIM)
    item_rows = _sc_gather(embedding, item_flat, DIM)
    nbrnw = _sc_gather(combined, flat, 128)

    neighbors = nbrnw[:, :S].reshape(1, B * L * S)
    nw = jax.lax.bitcast_convert_type(nbrnw[:, S:2 * S], f32)
    nw_col = nw.reshape(B * L * S, 1)

    neigh = _sc_gather(embedding, neighbors, DIM)

    h = h_rows.reshape(B, L, DIM)
    item_v = item_rows.reshape(B, L, DIM)

    a_stack = jnp.concatenate(
        [a0.T, a1.T, a2.T, a3.T, jnp.zeros((4, DIM), f32)], axis=0)

    BB1 = 16
    h_local, ev = pl.pallas_call(
        _local_body,
        grid=(B // BB1,),
        in_specs=[
            pl.BlockSpec((BB1, L, DIM), lambda i: (i, 0, 0)),
            pl.BlockSpec((BB1, L, L), lambda i: (i, 0, 0)),
            pl.BlockSpec((BB1, L, DIM), lambda i: (i, 0, 0)),
            pl.BlockSpec((BB1, L), lambda i: (i, 0)),
            pl.BlockSpec((8, DIM), lambda i: (0, 0)),
        ],
        out_specs=[
            pl.BlockSpec((BB1, L, DIM), lambda i: (i, 0, 0)),
            pl.BlockSpec((BB1, DIM), lambda i: (i, 0)),
        ],
        out_shape=[
            jax.ShapeDtypeStruct((B, L, DIM), f32),
            jax.ShapeDtypeStruct((B, DIM), f32),
        ],
    )(h, adj.astype(jnp.int32), item_v, mask_item.astype(jnp.int32), a_stack)

    nv = neigh.reshape(B, L * S, DIM)
    w1b = jnp.concatenate([g_w1[DIM:DIM + 1, :], jnp.zeros((7, DIM), f32)], axis=0)
    w2rep = jnp.broadcast_to(g_w2, (DIM, DIM))

    BB2 = 8
    output, s_global = pl.pallas_call(
        _global_body,
        grid=(B // BB2,),
        in_specs=[
            pl.BlockSpec((BB2, L * S, DIM), lambda i: (i, 0, 0)),
            pl.BlockSpec((BB2 * L * S, 1), lambda i: (i, 0)),
            pl.BlockSpec((BB2, DIM), lambda i: (i, 0)),
            pl.BlockSpec((BB2, L, DIM), lambda i: (i, 0, 0)),
            pl.BlockSpec((BB2, L, DIM), lambda i: (i, 0, 0)),
            pl.BlockSpec((DIM, DIM), lambda i: (0, 0)),
            pl.BlockSpec((8, DIM), lambda i: (0, 0)),
            pl.BlockSpec((DIM, DIM), lambda i: (0, 0)),
            pl.BlockSpec((DIM, DIM), lambda i: (0, 0)),
            pl.BlockSpec((DIM, DIM), lambda i: (0, 0)),
        ],
        out_specs=[
            pl.BlockSpec((BB2, L, DIM), lambda i: (i, 0, 0)),
            pl.BlockSpec((BB2, L, DIM), lambda i: (i, 0, 0)),
        ],
        out_shape=[
            jax.ShapeDtypeStruct((B, L, DIM), f32),
            jax.ShapeDtypeStruct((B, L, DIM), f32),
        ],
    )(nv, nw_col, ev, h, h_local,
      g_w1[:DIM, :], w1b, w2rep, g_w3[:DIM, :], g_w3[DIM:, :])

    return (output, s_global)


# speedup vs baseline: 2.4377x; 1.2615x over previous
import jax
import jax.numpy as jnp
from jax.experimental import pallas as pl
from jax.experimental.pallas import tpu as pltpu
from jax.experimental.pallas import tpu_sc as plsc

DIM = 128
B = 1024
L = 20
S = 12
LRELU = 0.2
NEG = -9e15

_GW = 128


def _leaky(x, slope=LRELU):
    return jnp.where(x >= 0, x, slope * x)


def _sc_gather(table, idx, width):
    n = idx.shape[1]
    mesh = plsc.VectorSubcoreMesh(core_axis_name="core", subcore_axis_name="subcore")

    @pl.kernel(out_type=jax.ShapeDtypeStruct((n, width), table.dtype), mesh=mesh)
    def gather_kernel(tbl_hbm, i_hbm, o_hbm):
        def body(i_vmem, o_vmem):
            pltpu.sync_copy(tbl_hbm.at[i_vmem.at[0]], o_vmem)

        pltpu.emit_pipeline(
            body,
            grid=(n // _GW,),
            in_specs=[pl.BlockSpec((1, _GW), lambda i: (0, i))],
            out_specs=[pl.BlockSpec((_GW, width), lambda i: (i, 0))],
            core_axis_name=("core", "subcore"),
            dimension_semantics=(pltpu.PARALLEL,),
        )(i_hbm, o_hbm)

    return gather_kernel(table, idx)


def _local_body(h_ref, ht_ref, adj_ref, item_ref, mask_ref, a_ref, hl_ref, ev_ref):
    bb = h_ref.shape[0]
    maskf = mask_ref[...].astype(jnp.float32)
    item_v = item_ref[...]
    s = jnp.sum(item_v * maskf[..., None], axis=1)
    ev_ref[...] = s / jnp.sum(maskf, axis=1, keepdims=True)

    a = a_ref[...]
    es = []
    for i in range(bb):
        hb = h_ref[i]
        hs = jnp.concatenate(
            [hb * a[0:1], hb * a[1:2], hb * a[2:3], hb * a[3:4]], axis=0
        )
        es.append(jax.lax.dot_general(
            hs, ht_ref[i], (((1,), (0,)), ((), ())),
            preferred_element_type=jnp.float32,
        ))
    e = _leaky(jnp.concatenate([x.reshape(1, 4, L, L) for x in es], axis=0))
    adjb = adj_ref[...]
    att = jnp.where(adjb == 1, e[:, 0], jnp.float32(NEG))
    att = jnp.where(adjb == 2, e[:, 1], att)
    att = jnp.where(adjb == 3, e[:, 2], att)
    att = jnp.where(adjb == 4, e[:, 3], att)
    att = att - jnp.max(att, axis=-1, keepdims=True)
    p = jnp.exp(att)
    att = p / jnp.sum(p, axis=-1, keepdims=True)
    for i in range(bb):
        hl_ref[i] = jax.lax.dot_general(
            att[i], h_ref[i], (((1,), (0,)), ((), ())),
            preferred_element_type=jnp.float32,
        )


def _global_body(nv_ref, nw_ref, ev_ref, h_ref, hl_ref,
                 w1a_ref, w1b_ref, w2_ref, w3h_ref, w3a_ref,
                 out_ref, sg_ref):
    bl = nv_ref.shape[1]
    bb = ev_ref.shape[0]
    m = S * bl
    nv = nv_ref[...]
    ev = ev_ref[...]
    ev_bl = jnp.broadcast_to(ev[:, None, :], (bb, L, DIM)).reshape(bl, DIM)
    scaled = (nv * ev_bl[None, :, :]).reshape(m, DIM)
    alp = jax.lax.dot_general(
        scaled, w1a_ref[...], (((1,), (0,)), ((), ())),
        preferred_element_type=jnp.float32,
    )
    alp = alp + nw_ref[...].reshape(m, 1) * w1b_ref[0:1, :]
    alp = _leaky(alp)
    al2 = jax.lax.dot_general(
        alp, w2_ref[...], (((1,), (0,)), ((), ())),
        preferred_element_type=jnp.float32,
    ).reshape(S, bl, DIM)
    mx = jnp.max(al2, axis=0)
    w = jnp.exp(al2 - mx[None, :, :])
    w = w / jnp.sum(w, axis=0)[None, :, :]
    agg = jnp.sum(w * nv, axis=0)
    out = jax.lax.dot_general(
        h_ref[...], w3h_ref[...], (((1,), (0,)), ((), ())),
        preferred_element_type=jnp.float32,
    ) + jax.lax.dot_general(
        agg, w3a_ref[...], (((1,), (0,)), ((), ())),
        preferred_element_type=jnp.float32,
    )
    sg = jnp.maximum(out, 0.0)
    sg_ref[...] = sg
    out_ref[...] = hl_ref[...] + sg


def kernel(inputs, adj, mask_item, item, embedding, a0, a1, a2, a3, g_w1, g_w2, g_w3, num_w, adj_all):
    f32 = jnp.float32
    flat = inputs.reshape(B * L).astype(jnp.int32)
    item_flat = item.reshape(B * L).astype(jnp.int32)
    both = jnp.concatenate([flat, item_flat]).reshape(1, 2 * B * L)

    nw_bits = jax.lax.bitcast_convert_type(num_w, jnp.int32)
    combined = jnp.concatenate(
        [adj_all.astype(jnp.int32), nw_bits,
         jnp.zeros((adj_all.shape[0], 128 - 2 * S), jnp.int32)], axis=1)

    hi_rows = _sc_gather(embedding, both, DIM)
    nbrnw = _sc_gather(combined, flat.reshape(1, B * L), 128)

    neighbors = nbrnw[:, :S].T.reshape(1, B * L * S)
    nw_col = jax.lax.bitcast_convert_type(nbrnw[:, S:2 * S], f32).T.reshape(S, B * L, 1)

    neigh = _sc_gather(embedding, neighbors, DIM)

    h_rows = hi_rows[:B * L]
    h = h_rows.reshape(B, L, DIM)
    ht = jnp.swapaxes(h, 1, 2)
    item_v = hi_rows[B * L:].reshape(B, L, DIM)

    a_stack = jnp.concatenate(
        [a0.T, a1.T, a2.T, a3.T, jnp.zeros((4, DIM), f32)], axis=0)

    BB1 = 16
    h_local, ev = pl.pallas_call(
        _local_body,
        grid=(B // BB1,),
        in_specs=[
            pl.BlockSpec((BB1, L, DIM), lambda i: (i, 0, 0)),
            pl.BlockSpec((BB1, DIM, L), lambda i: (i, 0, 0)),
            pl.BlockSpec((BB1, L, L), lambda i: (i, 0, 0)),
            pl.BlockSpec((BB1, L, DIM), lambda i: (i, 0, 0)),
            pl.BlockSpec((BB1, L), lambda i: (i, 0)),
            pl.BlockSpec((8, DIM), lambda i: (0, 0)),
        ],
        out_specs=[
            pl.BlockSpec((BB1, L, DIM), lambda i: (i, 0, 0)),
            pl.BlockSpec((BB1, DIM), lambda i: (i, 0)),
        ],
        out_shape=[
            jax.ShapeDtypeStruct((B, L, DIM), f32),
            jax.ShapeDtypeStruct((B, DIM), f32),
        ],
    )(h, ht, adj.astype(jnp.int32), item_v, mask_item.astype(jnp.int32), a_stack)

    nv = neigh.reshape(S, B * L, DIM)
    w1b = jnp.concatenate([g_w1[DIM:DIM + 1, :], jnp.zeros((7, DIM), f32)], axis=0)
    w2rep = jnp.broadcast_to(g_w2, (DIM, DIM))

    BB2 = 8
    BL2 = BB2 * L
    output, s_global = pl.pallas_call(
        _global_body,
        grid=(B // BB2,),
        in_specs=[
            pl.BlockSpec((S, BL2, DIM), lambda i: (0, i, 0)),
            pl.BlockSpec((S, BL2, 1), lambda i: (0, i, 0)),
            pl.BlockSpec((BB2, DIM), lambda i: (i, 0)),
            pl.BlockSpec((BL2, DIM), lambda i: (i, 0)),
            pl.BlockSpec((BL2, DIM), lambda i: (i, 0)),
            pl.BlockSpec((DIM, DIM), lambda i: (0, 0)),
            pl.BlockSpec((8, DIM), lambda i: (0, 0)),
            pl.BlockSpec((DIM, DIM), lambda i: (0, 0)),
            pl.BlockSpec((DIM, DIM), lambda i: (0, 0)),
            pl.BlockSpec((DIM, DIM), lambda i: (0, 0)),
        ],
        out_specs=[
            pl.BlockSpec((BL2, DIM), lambda i: (i, 0)),
            pl.BlockSpec((BL2, DIM), lambda i: (i, 0)),
        ],
        out_shape=[
            jax.ShapeDtypeStruct((B * L, DIM), f32),
            jax.ShapeDtypeStruct((B * L, DIM), f32),
        ],
    )(nv, nw_col, ev, h_rows, h_local.reshape(B * L, DIM),
      g_w1[:DIM, :], w1b, w2rep, g_w3[:DIM, :], g_w3[DIM:, :])

    return (output.reshape(B, L, DIM), s_global.reshape(B, L, DIM))


# speedup vs baseline: 3.0077x; 1.2338x over previous
import jax
import jax.numpy as jnp
from jax.experimental import pallas as pl
from jax.experimental.pallas import tpu as pltpu
from jax.experimental.pallas import tpu_sc as plsc

DIM = 128
B = 1024
L = 20
S = 12
LRELU = 0.2
NEG = -9e15

_GW = 128


def _leaky(x, slope=LRELU):
    return jnp.where(x >= 0, x, slope * x)


def _sc_gather(table, idx, width):
    n = idx.shape[1]
    mesh = plsc.VectorSubcoreMesh(core_axis_name="core", subcore_axis_name="subcore")

    @pl.kernel(out_type=jax.ShapeDtypeStruct((n, width), table.dtype), mesh=mesh)
    def gather_kernel(tbl_hbm, i_hbm, o_hbm):
        def body(i_vmem, o_vmem):
            pltpu.sync_copy(tbl_hbm.at[i_vmem.at[0]], o_vmem)

        pltpu.emit_pipeline(
            body,
            grid=(n // _GW,),
            in_specs=[pl.BlockSpec((1, _GW), lambda i: (0, i))],
            out_specs=[pl.BlockSpec((_GW, width), lambda i: (i, 0))],
            core_axis_name=("core", "subcore"),
            dimension_semantics=(pltpu.PARALLEL,),
        )(i_hbm, o_hbm)

    return gather_kernel(table, idx)


def _local_body(h_ref, ht_ref, adj_ref, item_ref, mask_ref, a_ref, hl_ref, ev_ref):
    bb = h_ref.shape[0]
    maskf = mask_ref[...].astype(jnp.float32)
    item_v = item_ref[...]
    s = jnp.sum(item_v * maskf[..., None], axis=1)
    ev_ref[...] = s / jnp.sum(maskf, axis=1, keepdims=True)

    a = a_ref[...]
    es = []
    for i in range(bb):
        hb = h_ref[i]
        hs = jnp.concatenate(
            [hb * a[0:1], hb * a[1:2], hb * a[2:3], hb * a[3:4]], axis=0
        )
        es.append(jax.lax.dot_general(
            hs, ht_ref[i], (((1,), (0,)), ((), ())),
            preferred_element_type=jnp.float32,
        ))
    e = _leaky(jnp.concatenate([x.reshape(1, 4, L, L) for x in es], axis=0))
    adjb = adj_ref[...]
    att = jnp.where(adjb == 1, e[:, 0], jnp.float32(NEG))
    att = jnp.where(adjb == 2, e[:, 1], att)
    att = jnp.where(adjb == 3, e[:, 2], att)
    att = jnp.where(adjb == 4, e[:, 3], att)
    att = att - jnp.max(att, axis=-1, keepdims=True)
    p = jnp.exp(att)
    att = p / jnp.sum(p, axis=-1, keepdims=True)
    for i in range(bb):
        hl_ref[i] = jax.lax.dot_general(
            att[i], h_ref[i], (((1,), (0,)), ((), ())),
            preferred_element_type=jnp.float32,
        )


def _global_body(nv_ref, nw_ref, ev_ref, h_ref, hl_ref,
                 w1a_ref, w1b_ref, w2_ref, w3h_ref, w3a_ref,
                 out_ref, sg_ref):
    bl = nv_ref.shape[1]
    bb = ev_ref.shape[0]
    m = S * bl
    nv = nv_ref[...]
    ev = ev_ref[...]
    ev_bl = jnp.broadcast_to(ev[:, None, :], (bb, L, DIM)).reshape(bl, DIM)
    scaled = (nv * ev_bl[None, :, :]).reshape(m, DIM)
    alp = jax.lax.dot_general(
        scaled, w1a_ref[...], (((1,), (0,)), ((), ())),
        preferred_element_type=jnp.float32,
    )
    nw_sm = jnp.swapaxes(nw_ref[...], 0, 1)
    alp = alp.reshape(S, bl, DIM) + nw_sm[:, :, None] * w1b_ref[0:1, :].reshape(1, 1, DIM)
    alp = alp.reshape(m, DIM)
    alp = _leaky(alp)
    al2 = jax.lax.dot_general(
        alp, w2_ref[...], (((1,), (0,)), ((), ())),
        preferred_element_type=jnp.float32,
    ).reshape(S, bl, DIM)
    mx = jnp.max(al2, axis=0)
    w = jnp.exp(al2 - mx[None, :, :])
    w = w / jnp.sum(w, axis=0)[None, :, :]
    agg = jnp.sum(w * nv, axis=0)
    out = jax.lax.dot_general(
        h_ref[...], w3h_ref[...], (((1,), (0,)), ((), ())),
        preferred_element_type=jnp.float32,
    ) + jax.lax.dot_general(
        agg, w3a_ref[...], (((1,), (0,)), ((), ())),
        preferred_element_type=jnp.float32,
    )
    sg = jnp.maximum(out, 0.0)
    sg_ref[...] = sg
    out_ref[...] = hl_ref[...] + sg


def kernel(inputs, adj, mask_item, item, embedding, a0, a1, a2, a3, g_w1, g_w2, g_w3, num_w, adj_all):
    f32 = jnp.float32
    flat = inputs.reshape(B * L).astype(jnp.int32)
    item_flat = item.reshape(B * L).astype(jnp.int32)
    both = jnp.concatenate([flat, item_flat]).reshape(1, 2 * B * L)

    nw_bits = jax.lax.bitcast_convert_type(num_w, jnp.int32)
    combined = jnp.concatenate(
        [adj_all.astype(jnp.int32), nw_bits,
         jnp.zeros((adj_all.shape[0], 128 - 2 * S), jnp.int32)], axis=1)

    hi_rows = _sc_gather(embedding, both, DIM)
    nbrnw = _sc_gather(combined, flat.reshape(1, B * L), 128)

    neighbors = nbrnw[:, :S].T.reshape(1, B * L * S)
    nw_nm = jax.lax.bitcast_convert_type(nbrnw[:, S:2 * S], f32)

    neigh = _sc_gather(embedding, neighbors, DIM)

    h_rows = hi_rows[:B * L]
    h = h_rows.reshape(B, L, DIM)
    ht = jnp.swapaxes(h, 1, 2)
    item_v = hi_rows[B * L:].reshape(B, L, DIM)

    a_stack = jnp.concatenate(
        [a0.T, a1.T, a2.T, a3.T, jnp.zeros((4, DIM), f32)], axis=0)

    BB1 = 16
    h_local, ev = pl.pallas_call(
        _local_body,
        grid=(B // BB1,),
        in_specs=[
            pl.BlockSpec((BB1, L, DIM), lambda i: (i, 0, 0)),
            pl.BlockSpec((BB1, DIM, L), lambda i: (i, 0, 0)),
            pl.BlockSpec((BB1, L, L), lambda i: (i, 0, 0)),
            pl.BlockSpec((BB1, L, DIM), lambda i: (i, 0, 0)),
            pl.BlockSpec((BB1, L), lambda i: (i, 0)),
            pl.BlockSpec((8, DIM), lambda i: (0, 0)),
        ],
        out_specs=[
            pl.BlockSpec((BB1, L, DIM), lambda i: (i, 0, 0)),
            pl.BlockSpec((BB1, DIM), lambda i: (i, 0)),
        ],
        out_shape=[
            jax.ShapeDtypeStruct((B, L, DIM), f32),
            jax.ShapeDtypeStruct((B, DIM), f32),
        ],
    )(h, ht, adj.astype(jnp.int32), item_v, mask_item.astype(jnp.int32), a_stack)

    nv = neigh.reshape(S, B * L, DIM)
    w1b = jnp.concatenate([g_w1[DIM:DIM + 1, :], jnp.zeros((7, DIM), f32)], axis=0)
    w2rep = jnp.broadcast_to(g_w2, (DIM, DIM))

    BB2 = 8
    BL2 = BB2 * L
    output, s_global = pl.pallas_call(
        _global_body,
        grid=(B // BB2,),
        in_specs=[
            pl.BlockSpec((S, BL2, DIM), lambda i: (0, i, 0)),
            pl.BlockSpec((BL2, S), lambda i: (i, 0)),
            pl.BlockSpec((BB2, DIM), lambda i: (i, 0)),
            pl.BlockSpec((BL2, DIM), lambda i: (i, 0)),
            pl.BlockSpec((BL2, DIM), lambda i: (i, 0)),
            pl.BlockSpec((DIM, DIM), lambda i: (0, 0)),
            pl.BlockSpec((8, DIM), lambda i: (0, 0)),
            pl.BlockSpec((DIM, DIM), lambda i: (0, 0)),
            pl.BlockSpec((DIM, DIM), lambda i: (0, 0)),
            pl.BlockSpec((DIM, DIM), lambda i: (0, 0)),
        ],
        out_specs=[
            pl.BlockSpec((BL2, DIM), lambda i: (i, 0)),
            pl.BlockSpec((BL2, DIM), lambda i: (i, 0)),
        ],
        out_shape=[
            jax.ShapeDtypeStruct((B * L, DIM), f32),
            jax.ShapeDtypeStruct((B * L, DIM), f32),
        ],
    )(nv, nw_nm, ev, h_rows, h_local.reshape(B * L, DIM),
      g_w1[:DIM, :], w1b, w2rep, g_w3[:DIM, :], g_w3[DIM:, :])

    return (output.reshape(B, L, DIM), s_global.reshape(B, L, DIM))


# speedup vs baseline: 3.5878x; 1.1929x over previous
import jax
import jax.numpy as jnp
from jax.experimental import pallas as pl
from jax.experimental.pallas import tpu as pltpu
from jax.experimental.pallas import tpu_sc as plsc

DIM = 128
B = 1024
L = 20
S = 12
LRELU = 0.2
NEG = -9e15

_GW = 128


def _leaky(x, slope=LRELU):
    return jnp.where(x >= 0, x, slope * x)


def _sc_gather(table, idx, width):
    n = idx.shape[1]
    mesh = plsc.VectorSubcoreMesh(core_axis_name="core", subcore_axis_name="subcore")

    @pl.kernel(out_type=jax.ShapeDtypeStruct((n, width), table.dtype), mesh=mesh)
    def gather_kernel(tbl_hbm, i_hbm, o_hbm):
        def body(i_vmem, o_vmem):
            pltpu.sync_copy(tbl_hbm.at[i_vmem.at[0]], o_vmem)

        pltpu.emit_pipeline(
            body,
            grid=(n // _GW,),
            in_specs=[pl.BlockSpec((1, _GW), lambda i: (0, i))],
            out_specs=[pl.BlockSpec((_GW, width), lambda i: (i, 0))],
            core_axis_name=("core", "subcore"),
            dimension_semantics=(pltpu.PARALLEL,),
        )(i_hbm, o_hbm)

    return gather_kernel(table, idx)


def _local_body(h_ref, adj_ref, item_ref, mask_ref, a_ref, hl_ref, ev_ref):
    bb = h_ref.shape[0]
    maskf = mask_ref[...].astype(jnp.float32)
    item_v = item_ref[...]
    s = jnp.sum(item_v * maskf[..., None], axis=1)
    ev_ref[...] = s / jnp.sum(maskf, axis=1, keepdims=True)

    a = a_ref[...]
    es = []
    for i in range(bb):
        hb = h_ref[i]
        hs = jnp.concatenate(
            [hb * a[0:1], hb * a[1:2], hb * a[2:3], hb * a[3:4]], axis=0
        )
        es.append(jax.lax.dot_general(
            hs, hb, (((1,), (1,)), ((), ())),
            preferred_element_type=jnp.float32,
        ))
    e = _leaky(jnp.concatenate([x.reshape(1, 4, L, L) for x in es], axis=0))
    adjb = adj_ref[...]
    att = jnp.where(adjb == 1, e[:, 0], jnp.float32(NEG))
    att = jnp.where(adjb == 2, e[:, 1], att)
    att = jnp.where(adjb == 3, e[:, 2], att)
    att = jnp.where(adjb == 4, e[:, 3], att)
    att = att - jnp.max(att, axis=-1, keepdims=True)
    p = jnp.exp(att)
    att = p / jnp.sum(p, axis=-1, keepdims=True)
    for i in range(bb):
        hl_ref[i] = jax.lax.dot_general(
            att[i], h_ref[i], (((1,), (0,)), ((), ())),
            preferred_element_type=jnp.float32,
        )


def _global_body(nv_ref, nw_ref, ev_ref, h_ref, hl_ref,
                 w1a_ref, w1b_ref, w2_ref, w3h_ref, w3a_ref,
                 out_ref, sg_ref):
    bl = nv_ref.shape[1]
    bb = ev_ref.shape[0]
    m = S * bl
    nv = nv_ref[...]
    ev = ev_ref[...]
    ev_bl = jnp.broadcast_to(ev[:, None, :], (bb, L, DIM)).reshape(bl, DIM)
    scaled = (nv * ev_bl[None, :, :]).reshape(m, DIM)
    alp = jax.lax.dot_general(
        scaled, w1a_ref[...], (((1,), (0,)), ((), ())),
        preferred_element_type=jnp.float32,
    )
    nw_sm = jnp.swapaxes(nw_ref[...], 0, 1)
    alp = alp.reshape(S, bl, DIM) + nw_sm[:, :, None] * w1b_ref[0:1, :].reshape(1, 1, DIM)
    alp = alp.reshape(m, DIM)
    alp = _leaky(alp)
    al2 = jax.lax.dot_general(
        alp, w2_ref[...], (((1,), (0,)), ((), ())),
        preferred_element_type=jnp.float32,
    ).reshape(S, bl, DIM)
    mx = jnp.max(al2, axis=0)
    w = jnp.exp(al2 - mx[None, :, :])
    w = w / jnp.sum(w, axis=0)[None, :, :]
    agg = jnp.sum(w * nv, axis=0)
    out = jax.lax.dot_general(
        h_ref[...].reshape(bl, DIM), w3h_ref[...], (((1,), (0,)), ((), ())),
        preferred_element_type=jnp.float32,
    ) + jax.lax.dot_general(
        agg, w3a_ref[...], (((1,), (0,)), ((), ())),
        preferred_element_type=jnp.float32,
    )
    sg = jnp.maximum(out, 0.0).reshape(bb, L, DIM)
    sg_ref[...] = sg
    out_ref[...] = hl_ref[...] + sg


def kernel(inputs, adj, mask_item, item, embedding, a0, a1, a2, a3, g_w1, g_w2, g_w3, num_w, adj_all):
    f32 = jnp.float32
    flat = inputs.reshape(B * L).astype(jnp.int32)
    item_flat = item.reshape(B * L).astype(jnp.int32)
    both = jnp.concatenate([flat, item_flat]).reshape(1, 2 * B * L)

    n_node = adj_all.shape[0]
    nw_bits = jax.lax.bitcast_convert_type(num_w, jnp.int32)
    combined = jnp.concatenate(
        [adj_all.astype(jnp.int32), nw_bits,
         jnp.zeros((n_node, 128 - 2 * S), jnp.int32)], axis=1)

    hi_rows = _sc_gather(embedding, both, DIM)
    nbrnw = _sc_gather(combined, flat.reshape(1, B * L), 128)

    neighbors = nbrnw[:, :S].T.reshape(1, B * L * S)
    nw_nm = jax.lax.bitcast_convert_type(nbrnw[:, S:2 * S], f32)

    neigh = _sc_gather(embedding, neighbors, DIM)

    h = hi_rows[:B * L].reshape(B, L, DIM)
    item_v = hi_rows[B * L:].reshape(B, L, DIM)

    a_stack = jnp.concatenate(
        [a0.T, a1.T, a2.T, a3.T, jnp.zeros((4, DIM), f32)], axis=0)

    BB1 = 16
    h_local, ev = pl.pallas_call(
        _local_body,
        grid=(B // BB1,),
        in_specs=[
            pl.BlockSpec((BB1, L, DIM), lambda i: (i, 0, 0)),
            pl.BlockSpec((BB1, L, L), lambda i: (i, 0, 0)),
            pl.BlockSpec((BB1, L, DIM), lambda i: (i, 0, 0)),
            pl.BlockSpec((BB1, L), lambda i: (i, 0)),
            pl.BlockSpec((8, DIM), lambda i: (0, 0)),
        ],
        out_specs=[
            pl.BlockSpec((BB1, L, DIM), lambda i: (i, 0, 0)),
            pl.BlockSpec((BB1, DIM), lambda i: (i, 0)),
        ],
        out_shape=[
            jax.ShapeDtypeStruct((B, L, DIM), f32),
            jax.ShapeDtypeStruct((B, DIM), f32),
        ],
    )(h, adj.astype(jnp.int32), item_v, mask_item.astype(jnp.int32), a_stack)

    nv = neigh.reshape(S, B * L, DIM)
    w1b = jnp.concatenate([g_w1[DIM:DIM + 1, :], jnp.zeros((7, DIM), f32)], axis=0)
    w2rep = jnp.broadcast_to(g_w2, (DIM, DIM))

    BB2 = 16
    BL2 = BB2 * L
    output, s_global = pl.pallas_call(
        _global_body,
        grid=(B // BB2,),
        in_specs=[
            pl.BlockSpec((S, BL2, DIM), lambda i: (0, i, 0)),
            pl.BlockSpec((BL2, S), lambda i: (i, 0)),
            pl.BlockSpec((BB2, DIM), lambda i: (i, 0)),
            pl.BlockSpec((BB2, L, DIM), lambda i: (i, 0, 0)),
            pl.BlockSpec((BB2, L, DIM), lambda i: (i, 0, 0)),
            pl.BlockSpec((DIM, DIM), lambda i: (0, 0)),
            pl.BlockSpec((8, DIM), lambda i: (0, 0)),
            pl.BlockSpec((DIM, DIM), lambda i: (0, 0)),
            pl.BlockSpec((DIM, DIM), lambda i: (0, 0)),
            pl.BlockSpec((DIM, DIM), lambda i: (0, 0)),
        ],
        out_specs=[
            pl.BlockSpec((BB2, L, DIM), lambda i: (i, 0, 0)),
            pl.BlockSpec((BB2, L, DIM), lambda i: (i, 0, 0)),
        ],
        out_shape=[
            jax.ShapeDtypeStruct((B, L, DIM), f32),
            jax.ShapeDtypeStruct((B, L, DIM), f32),
        ],
    )(nv, nw_nm, ev, h, h_local,
      g_w1[:DIM, :], w1b, w2rep, g_w3[:DIM, :], g_w3[DIM:, :])

    return (output, s_global)


# speedup vs baseline: 3.7384x; 1.0420x over previous
import jax
import jax.numpy as jnp
from jax.experimental import pallas as pl
from jax.experimental.pallas import tpu as pltpu
from jax.experimental.pallas import tpu_sc as plsc

DIM = 128
B = 1024
L = 20
S = 12
LRELU = 0.2
NEG = -9e15

_GW = 128


def _leaky(x, slope=LRELU):
    return jnp.where(x >= 0, x, slope * x)


def _sc_gather(table, idx, width):
    n = idx.shape[1]
    mesh = plsc.VectorSubcoreMesh(core_axis_name="core", subcore_axis_name="subcore")

    @pl.kernel(out_type=jax.ShapeDtypeStruct((n, width), table.dtype), mesh=mesh)
    def gather_kernel(tbl_hbm, i_hbm, o_hbm):
        def body(i_vmem, o_vmem):
            pltpu.sync_copy(tbl_hbm.at[i_vmem.at[0]], o_vmem)

        pltpu.emit_pipeline(
            body,
            grid=(n // _GW,),
            in_specs=[pl.BlockSpec((1, _GW), lambda i: (0, i))],
            out_specs=[pl.BlockSpec((_GW, width), lambda i: (i, 0))],
            core_axis_name=("core", "subcore"),
            dimension_semantics=(pltpu.PARALLEL,),
        )(i_hbm, o_hbm)

    return gather_kernel(table, idx)


def _table_body(adj_ref, nwb_ref, out_ref):
    wn = adj_ref.shape[0]
    out_ref[...] = jnp.concatenate(
        [adj_ref[...], nwb_ref[...],
         jnp.zeros((wn, 128 - 2 * S), jnp.int32)], axis=1)


def _build_table(adj_i32, nw_bits):
    n_node = adj_i32.shape[0]
    wn = 4000
    return pl.pallas_call(
        _table_body,
        grid=(n_node // wn,),
        in_specs=[
            pl.BlockSpec((wn, S), lambda i: (i, 0)),
            pl.BlockSpec((wn, S), lambda i: (i, 0)),
        ],
        out_specs=pl.BlockSpec((wn, 128), lambda i: (i, 0)),
        out_shape=jax.ShapeDtypeStruct((n_node, 128), jnp.int32),
    )(adj_i32, nw_bits)


def _local_body(h_ref, adj_ref, item_ref, mask_ref, a_ref, hl_ref, ev_ref):
    bb = h_ref.shape[0]
    maskf = mask_ref[...].astype(jnp.float32)
    item_v = item_ref[...]
    s = jnp.sum(item_v * maskf[..., None], axis=1)
    ev_ref[...] = s / jnp.sum(maskf, axis=1, keepdims=True)

    a = a_ref[...]
    es = []
    for i in range(bb):
        hb = h_ref[i]
        hs = jnp.concatenate(
            [hb * a[0:1], hb * a[1:2], hb * a[2:3], hb * a[3:4]], axis=0
        )
        es.append(jax.lax.dot_general(
            hs, hb, (((1,), (1,)), ((), ())),
            preferred_element_type=jnp.float32,
        ))
    e = _leaky(jnp.concatenate([x.reshape(1, 4, L, L) for x in es], axis=0))
    adjb = adj_ref[...]
    att = jnp.where(adjb == 1, e[:, 0], jnp.float32(NEG))
    att = jnp.where(adjb == 2, e[:, 1], att)
    att = jnp.where(adjb == 3, e[:, 2], att)
    att = jnp.where(adjb == 4, e[:, 3], att)
    att = att - jnp.max(att, axis=-1, keepdims=True)
    p = jnp.exp(att)
    att = p / jnp.sum(p, axis=-1, keepdims=True)
    for i in range(bb):
        hl_ref[i] = jax.lax.dot_general(
            att[i], h_ref[i], (((1,), (0,)), ((), ())),
            preferred_element_type=jnp.float32,
        )


def _global_body(nv_ref, nw_ref, ev_ref, h_ref, hl_ref,
                 w1a_ref, w1b_ref, w2_ref, w3h_ref, w3a_ref,
                 out_ref, sg_ref):
    bl = nv_ref.shape[1]
    bb = ev_ref.shape[0]
    m = S * bl
    nv = nv_ref[...]
    ev = ev_ref[...]
    ev_bl = jnp.broadcast_to(ev[:, None, :], (bb, L, DIM)).reshape(bl, DIM)
    scaled = (nv * ev_bl[None, :, :]).reshape(m, DIM)
    alp = jax.lax.dot_general(
        scaled, w1a_ref[...], (((1,), (0,)), ((), ())),
        preferred_element_type=jnp.float32,
    )
    nw_sm = jnp.swapaxes(nw_ref[...], 0, 1)
    alp = alp.reshape(S, bl, DIM) + nw_sm[:, :, None] * w1b_ref[0:1, :].reshape(1, 1, DIM)
    alp = alp.reshape(m, DIM)
    alp = _leaky(alp)
    al2 = jax.lax.dot_general(
        alp, w2_ref[...], (((1,), (0,)), ((), ())),
        preferred_element_type=jnp.float32,
    ).reshape(S, bl, DIM)
    mx = jnp.max(al2, axis=0)
    w = jnp.exp(al2 - mx[None, :, :])
    w = w / jnp.sum(w, axis=0)[None, :, :]
    agg = jnp.sum(w * nv, axis=0)
    out = jax.lax.dot_general(
        h_ref[...].reshape(bl, DIM), w3h_ref[...], (((1,), (0,)), ((), ())),
        preferred_element_type=jnp.float32,
    ) + jax.lax.dot_general(
        agg, w3a_ref[...], (((1,), (0,)), ((), ())),
        preferred_element_type=jnp.float32,
    )
    sg = jnp.maximum(out, 0.0).reshape(bb, L, DIM)
    sg_ref[...] = sg
    out_ref[...] = hl_ref[...] + sg


def kernel(inputs, adj, mask_item, item, embedding, a0, a1, a2, a3, g_w1, g_w2, g_w3, num_w, adj_all):
    f32 = jnp.float32
    flat = inputs.reshape(B * L).astype(jnp.int32)
    item_flat = item.reshape(B * L).astype(jnp.int32)
    both = jnp.concatenate([flat, item_flat]).reshape(1, 2 * B * L)

    nw_bits = jax.lax.bitcast_convert_type(num_w, jnp.int32)
    combined = _build_table(adj_all.astype(jnp.int32), nw_bits)

    hi_rows = _sc_gather(embedding, both, DIM)
    nbrnw = _sc_gather(combined, flat.reshape(1, B * L), 128)

    neighbors = nbrnw[:, :S].T.reshape(1, B * L * S)
    nw_nm = jax.lax.bitcast_convert_type(nbrnw[:, S:2 * S], f32)

    neigh = _sc_gather(embedding, neighbors, DIM)

    h = hi_rows[:B * L].reshape(B, L, DIM)
    item_v = hi_rows[B * L:].reshape(B, L, DIM)

    a_stack = jnp.concatenate(
        [a0.T, a1.T, a2.T, a3.T, jnp.zeros((4, DIM), f32)], axis=0)

    BB1 = 16
    h_local, ev = pl.pallas_call(
        _local_body,
        grid=(B // BB1,),
        in_specs=[
            pl.BlockSpec((BB1, L, DIM), lambda i: (i, 0, 0)),
            pl.BlockSpec((BB1, L, L), lambda i: (i, 0, 0)),
            pl.BlockSpec((BB1, L, DIM), lambda i: (i, 0, 0)),
            pl.BlockSpec((BB1, L), lambda i: (i, 0)),
            pl.BlockSpec((8, DIM), lambda i: (0, 0)),
        ],
        out_specs=[
            pl.BlockSpec((BB1, L, DIM), lambda i: (i, 0, 0)),
            pl.BlockSpec((BB1, DIM), lambda i: (i, 0)),
        ],
        out_shape=[
            jax.ShapeDtypeStruct((B, L, DIM), f32),
            jax.ShapeDtypeStruct((B, DIM), f32),
        ],
    )(h, adj.astype(jnp.int32), item_v, mask_item.astype(jnp.int32), a_stack)

    nv = neigh.reshape(S, B * L, DIM)
    w1b = jnp.concatenate([g_w1[DIM:DIM + 1, :], jnp.zeros((7, DIM), f32)], axis=0)
    w2rep = jnp.broadcast_to(g_w2, (DIM, DIM))

    BB2 = 32
    BL2 = BB2 * L
    output, s_global = pl.pallas_call(
        _global_body,
        grid=(B // BB2,),
        in_specs=[
            pl.BlockSpec((S, BL2, DIM), lambda i: (0, i, 0)),
            pl.BlockSpec((BL2, S), lambda i: (i, 0)),
            pl.BlockSpec((BB2, DIM), lambda i: (i, 0)),
            pl.BlockSpec((BB2, L, DIM), lambda i: (i, 0, 0)),
            pl.BlockSpec((BB2, L, DIM), lambda i: (i, 0, 0)),
            pl.BlockSpec((DIM, DIM), lambda i: (0, 0)),
            pl.BlockSpec((8, DIM), lambda i: (0, 0)),
            pl.BlockSpec((DIM, DIM), lambda i: (0, 0)),
            pl.BlockSpec((DIM, DIM), lambda i: (0, 0)),
            pl.BlockSpec((DIM, DIM), lambda i: (0, 0)),
        ],
        out_specs=[
            pl.BlockSpec((BB2, L, DIM), lambda i: (i, 0, 0)),
            pl.BlockSpec((BB2, L, DIM), lambda i: (i, 0, 0)),
        ],
        out_shape=[
            jax.ShapeDtypeStruct((B, L, DIM), f32),
            jax.ShapeDtypeStruct((B, L, DIM), f32),
        ],
    )(nv, nw_nm, ev, h, h_local,
      g_w1[:DIM, :], w1b, w2rep, g_w3[:DIM, :], g_w3[DIM:, :])

    return (output, s_global)
